# baseline (speedup 1.0000x reference)
import functools

import jax
import jax.numpy as jnp
from jax import lax
from jax.experimental import pallas as pl
from jax.experimental.pallas import tpu as pltpu
from jax.experimental.pallas import tpu_sc as plsc

_N = 50000
_E = 800000
_CH = 128
_EP = 819200
_NCH = _EP // _CH
_ACC = 50176
_ZSTRIPE = _ACC // 16
_ZB = 784
_WSTRIPE = 3128
_NOUT = 50048
_BLK = 1000


def _l1_body(x_ref, w_ref, alx_ref, arx_ref, fse_ref, erx_ref):
    f = jnp.dot(x_ref[...], w_ref[...], preferred_element_type=jnp.float32)
    elxx = jnp.dot(f, alx_ref[...], preferred_element_type=jnp.float32)
    erxx = jnp.dot(f, arx_ref[...], preferred_element_type=jnp.float32)
    for q in range(4):
        fse_ref[q] = jnp.concatenate(
            [f[:, 16 * q:16 * q + 16], elxx[:, 16 * q:16 * q + 16]], axis=1)
        erx_ref[q] = erxx[:, 16 * q:16 * q + 16]


def _layer1(x, w1, alx, arx):
    n = x.shape[0]
    k = x.shape[1]
    return pl.pallas_call(
        _l1_body,
        grid=(n // _BLK,),
        in_specs=[
            pl.BlockSpec((_BLK, k), lambda i: (i, 0)),
            pl.BlockSpec((k, 64), lambda i: (0, 0)),
            pl.BlockSpec((64, 64), lambda i: (0, 0)),
            pl.BlockSpec((64, 64), lambda i: (0, 0)),
        ],
        out_specs=[
            pl.BlockSpec((4, _BLK, 32), lambda i: (0, i, 0)),
            pl.BlockSpec((4, _BLK, 16), lambda i: (0, i, 0)),
        ],
        out_shape=[
            jax.ShapeDtypeStruct((4, n, 32), jnp.float32),
            jax.ShapeDtypeStruct((4, n, 16), jnp.float32),
        ],
    )(x, w1, alx, arx)


def _edge1_body(src_hbm, dst_hbm, fse, erx, nd_out,
                sacc, wbuf, nbuf,
                sraw0, sraw1, draw0, draw1,
                sq0, sq1, dq0, dq1,
                dss0, dss1,
                gs0, gs1, gd0, gd1, upd0, upd1,
                sl0, sl1, sg0, sg1):
    cid = lax.axis_index("c")
    sid = lax.axis_index("s")
    sraw = [sraw0, sraw1]
    draw = [draw0, draw1]
    sq = [sq0, sq1]
    dq = [dq0, dq1]
    dss = [dss0, dss1]
    gs = [gs0, gs1]
    gd = [gd0, gd1]
    upd = [upd0, upd1]
    sl = [sl0, sl1]
    sg = [sg0, sg1]
    nch = _NCH // 16

    def zero_upd0(i, _):
        upd0[i] = jnp.zeros((32,), jnp.float32)
        return 0

    def issue_linear(ch, p):
        off = ch * _CH
        pltpu.make_async_copy(src_hbm.at[pl.ds(off, _CH)], sraw[p], sl[p]).start()
        pltpu.make_async_copy(dst_hbm.at[pl.ds(off, _CH)], draw[p], sl[p]).start()

    def wait_linear(p):
        pltpu.make_async_copy(src_hbm.at[pl.ds(0, _CH)], sraw[p], sl[p]).wait()
        pltpu.make_async_copy(dst_hbm.at[pl.ds(0, _CH)], draw[p], sl[p]).wait()

    def adjust(p, slot, qn):
        for l in range(8):
            s_ = pl.ds(l * 16, 16)
            s = sraw[p][s_]
            d = draw[p][s_]
            sq[p][s_] = s + qn
            dq[p][s_] = jnp.minimum(d, _N - 1) + qn
            dss[slot][s_] = d

    def issue_gathers(p):
        pltpu.make_async_copy(fse.at[sq[p]], gs[p], sg[p]).start()
        pltpu.make_async_copy(erx.at[dq[p]], gd[p], sg[p]).start()

    def wait_gathers(p):
        pltpu.make_async_copy(fse.at[sq[p]], gs[p], sg[p]).wait()
        pltpu.make_async_copy(erx.at[dq[p]], gd[p], sg[p]).wait()

    def compute(p):
        def rowblk(i, _):
            for u in range(4):
                r = i * 4 + u
                fe = gs[p][r, pl.ds(0, 16)]
                el = gs[p][r, pl.ds(16, 16)]
                e = el + gd[p][r]
                ex = jnp.exp(jnp.maximum(e, e * 0.2))
                upd[p][r, pl.ds(0, 16)] = fe * ex
                upd[p][r, pl.ds(16, 16)] = ex
            return 0

        lax.fori_loop(0, _CH // 4, rowblk, 0)

    base = sid * nch
    zbase = sid * _ZSTRIPE
    wb = sid * _WSTRIPE

    def nrow(i, _):
        nu = wbuf[i, pl.ds(0, 16)]
        de = wbuf[i, pl.ds(16, 16)]
        nbuf[i] = nu / jnp.maximum(de, 1e-9)
        return 0

    def step(i, p, qn):
        @pl.when(i + 2 < nch)
        def _():
            issue_linear(base + i + 2, p)

        @pl.when(i + 1 < nch)
        def _():
            wait_linear(1 - p)
            adjust(1 - p, 1 - p, qn)
            issue_gathers(1 - p)

        wait_gathers(p)
        compute(p)
        pltpu.sync_copy(upd[p], sacc.at[dss[p]], add=True)

    def round_body(g, _):
        qn = (cid * 2 + g) * _N
        q = cid * 2 + g
        lax.fori_loop(0, _CH, zero_upd0, 0)

        def zrow(z, c):
            pltpu.sync_copy(upd0, sacc.at[pl.ds(zbase + z * _CH, _CH)])
            return c

        lax.fori_loop(0, 24, zrow, 0)
        pltpu.sync_copy(upd0.at[pl.ds(0, 64)],
                        sacc.at[pl.ds(zbase + 24 * _CH, 64)])
        plsc.subcore_barrier()

        issue_linear(base, 0)
        wait_linear(0)
        adjust(0, 0, qn)
        issue_gathers(0)
        issue_linear(base + 1, 1)

        def outer(i2, c):
            step(i2 * 2, 0, qn)
            step(i2 * 2 + 1, 1, qn)
            return c

        lax.fori_loop(0, nch // 2, outer, 0)
        plsc.subcore_barrier()

        def wrow(z, c):
            roff = wb + z * _CH
            pltpu.sync_copy(sacc.at[pl.ds(roff, _CH)], wbuf)
            lax.fori_loop(0, _CH, nrow, 0)
            pltpu.sync_copy(nbuf, nd_out.at[q, pl.ds(roff, _CH)])
            return c

        lax.fori_loop(0, 24, wrow, 0)
        roff = wb + 24 * _CH
        pltpu.sync_copy(sacc.at[pl.ds(roff, 56)], wbuf.at[pl.ds(0, 56)])
        lax.fori_loop(0, 56, nrow, 0)
        pltpu.sync_copy(nbuf.at[pl.ds(0, 56)], nd_out.at[q, pl.ds(roff, 56)])
        plsc.subcore_barrier()
        return 0

    lax.fori_loop(0, 2, round_body, 0)


def _edge1(src_p, dst_p, fse, erx):
    mesh = plsc.VectorSubcoreMesh(core_axis_name="c", subcore_axis_name="s")
    f32 = jnp.float32
    i32 = jnp.int32
    idxbuf = pltpu.VMEM((_CH,), i32)
    kern = pl.kernel(
        _edge1_body,
        out_type=jax.ShapeDtypeStruct((4, _NOUT, 16), f32),
        mesh=mesh,
        compiler_params=pltpu.CompilerParams(use_tc_tiling_on_sc=False),
        scratch_types=[
            pltpu.VMEM_SHARED((_ACC, 32), f32),
            pltpu.VMEM((_CH, 32), f32),
            pltpu.VMEM((_CH, 16), f32),
            idxbuf, idxbuf, idxbuf, idxbuf,
            idxbuf, idxbuf, idxbuf, idxbuf,
            idxbuf, idxbuf,
            pltpu.VMEM((_CH, 32), f32), pltpu.VMEM((_CH, 32), f32),
            pltpu.VMEM((_CH, 16), f32), pltpu.VMEM((_CH, 16), f32),
            pltpu.VMEM((_CH, 32), f32), pltpu.VMEM((_CH, 32), f32),
            pltpu.SemaphoreType.DMA, pltpu.SemaphoreType.DMA,
            pltpu.SemaphoreType.DMA, pltpu.SemaphoreType.DMA,
        ],
    )
    return kern(src_p, dst_p, fse, erx)


def _edge3_body(src_hbm, dst_hbm, t3s, t3d, out3,
                acc,
                sraw0, sraw1, draw0, draw1,
                sq0, sq1, dq0, dq1,
                dss0, dss1,
                gs0, gs1, gd0, gd1, upd0, upd1,
                sl0, sl1, sg0, sg1):
    cid = lax.axis_index("c")
    sid = lax.axis_index("s")
    sraw = [sraw0, sraw1]
    draw = [draw0, draw1]
    sq = [sq0, sq1]
    dq = [dq0, dq1]
    dss = [dss0, dss1]
    gs = [gs0, gs1]
    gd = [gd0, gd1]
    upd = [upd0, upd1]
    sl = [sl0, sl1]
    sg = [sg0, sg1]
    nch = _NCH // 32
    wid = cid * 16 + sid

    def zero_upd0(i, _):
        upd0[i] = jnp.zeros((16,), jnp.float32)
        return 0

    lax.fori_loop(0, _CH, zero_upd0, 0)

    def issue_linear(ch, p):
        off = ch * _CH
        pltpu.make_async_copy(src_hbm.at[pl.ds(off, _CH)], sraw[p], sl[p]).start()
        pltpu.make_async_copy(dst_hbm.at[pl.ds(off, _CH)], draw[p], sl[p]).start()

    def wait_linear(p):
        pltpu.make_async_copy(src_hbm.at[pl.ds(0, _CH)], sraw[p], sl[p]).wait()
        pltpu.make_async_copy(dst_hbm.at[pl.ds(0, _CH)], draw[p], sl[p]).wait()

    def adjust(p, slot):
        for l in range(8):
            s_ = pl.ds(l * 16, 16)
            d = draw[p][s_]
            sq[p][s_] = sraw[p][s_]
            dq[p][s_] = jnp.minimum(d, _N - 1)
            dss[slot][s_] = d

    def issue_gathers(p):
        pltpu.make_async_copy(t3s.at[sq[p]], gs[p], sg[p]).start()
        pltpu.make_async_copy(t3d.at[dq[p]], gd[p], sg[p]).start()

    def wait_gathers(p):
        pltpu.make_async_copy(t3s.at[sq[p]], gs[p], sg[p]).wait()
        pltpu.make_async_copy(t3d.at[dq[p]], gd[p], sg[p]).wait()

    def compute(p):
        def rowblk(i, _):
            for u in range(4):
                r = i * 4 + u
                s = gs[p][r]
                e = s + gd[p][r]
                ex = jnp.exp(jnp.maximum(e, e * 0.2))
                exv = jnp.broadcast_to(ex[8], (16,))
                upd[p][r] = s * exv
            return 0

        lax.fori_loop(0, _CH // 4, rowblk, 0)

    zbase = sid * _ZSTRIPE

    def zrow(z, c):
        pltpu.sync_copy(upd0, acc.at[pl.ds(zbase + z * _CH, _CH)])
        return c

    lax.fori_loop(0, 24, zrow, 0)
    pltpu.sync_copy(upd0.at[pl.ds(0, 64)],
                    acc.at[pl.ds(zbase + 24 * _CH, 64)])
    plsc.subcore_barrier()

    base = wid * nch

    def step(i, p):
        @pl.when(i + 2 < nch)
        def _():
            issue_linear(base + i + 2, p)

        @pl.when(i + 1 < nch)
        def _():
            wait_linear(1 - p)
            adjust(1 - p, 1 - p)
            issue_gathers(1 - p)

        wait_gathers(p)
        compute(p)
        pltpu.sync_copy(upd[p], acc.at[dss[p]], add=True)

    issue_linear(base, 0)
    wait_linear(0)
    adjust(0, 0)
    issue_gathers(0)
    issue_linear(base + 1, 1)

    def outer(i2, c):
        step(i2 * 2, 0)
        step(i2 * 2 + 1, 1)
        return c

    lax.fori_loop(0, nch // 2, outer, 0)
    plsc.subcore_barrier()

    wb = sid * _ZSTRIPE
    pltpu.sync_copy(acc.at[pl.ds(wb, _ZSTRIPE)],
                    out3.at[cid, pl.ds(wb, _ZSTRIPE)])


def _edge3(src_p, dst_p, t3s, t3d):
    mesh = plsc.VectorSubcoreMesh(core_axis_name="c", subcore_axis_name="s")
    f32 = jnp.float32
    i32 = jnp.int32
    idxbuf = pltpu.VMEM((_CH,), i32)
    rowbuf = pltpu.VMEM((_CH, 16), f32)
    kern = pl.kernel(
        _edge3_body,
        out_type=jax.ShapeDtypeStruct((2, _ACC, 16), f32),
        mesh=mesh,
        compiler_params=pltpu.CompilerParams(use_tc_tiling_on_sc=False),
        scratch_types=[
            pltpu.VMEM_SHARED((_ACC, 16), f32),
            idxbuf, idxbuf, idxbuf, idxbuf,
            idxbuf, idxbuf, idxbuf, idxbuf,
            idxbuf, idxbuf,
            rowbuf, rowbuf, rowbuf, rowbuf, rowbuf, rowbuf,
            pltpu.SemaphoreType.DMA, pltpu.SemaphoreType.DMA,
            pltpu.SemaphoreType.DMA, pltpu.SemaphoreType.DMA,
        ],
    )
    return kern(src_p, dst_p, t3s, t3d)


def _mhi_body(nd_ref, b1_ref, bm_mat_ref, bmt_ref, a1_ref, v2_ref,
              r_ref, s_ref, w3_ref, al3_ref, ar3_ref, t3s_ref, t3d_ref):
    nd = nd_ref[...]
    rst = jnp.concatenate([nd[0], nd[1], nd[2], nd[3]], axis=-1)
    h = jnp.maximum(rst + b1_ref[...], 0.0)
    x2 = jnp.dot(h, bm_mat_ref[...], preferred_element_type=jnp.float32)
    x2 = x2 + bmt_ref[...]
    s1 = jnp.dot(x2, a1_ref[...], preferred_element_type=jnp.float32)
    s2 = jnp.dot(x2, v2_ref[...], preferred_element_type=jnp.float32)
    e = jnp.maximum(s1 + s2, 0.0)
    m = jnp.max(e, axis=1, keepdims=True)
    ex = jnp.exp(e - m)
    alpha = ex / jnp.sum(ex, axis=1, keepdims=True)
    alf = jnp.dot(alpha, r_ref[...], preferred_element_type=jnp.float32)
    h2 = jnp.dot(h * alf, s_ref[...], preferred_element_type=jnp.float32)
    f3 = jnp.dot(h2, w3_ref[...], preferred_element_type=jnp.float32)
    el3 = jnp.dot(f3, al3_ref[...], preferred_element_type=jnp.float32)
    er3 = jnp.dot(f3, ar3_ref[...], preferred_element_type=jnp.float32)
    blk = f3.shape[0]
    one = jnp.ones((blk, 1), jnp.float32)
    t3s_ref[...] = jnp.concatenate(
        [f3, one, jnp.broadcast_to(el3, (blk, 8))], axis=1)
    t3d_ref[...] = jnp.concatenate(
        [jnp.zeros((blk, 8), jnp.float32), jnp.broadcast_to(er3, (blk, 8))],
        axis=1)


def _mhi_layer(nd, b1, bm_mat, bmt, a1_mat, v2, r_mat, s_mat, w3, al3, ar3):
    full = lambda shape: pl.BlockSpec(shape, lambda i: tuple(0 for _ in shape))
    row = lambda c: pl.BlockSpec((_BLK, c), lambda i: (i, 0))
    return pl.pallas_call(
        _mhi_body,
        grid=(_N // _BLK,),
        in_specs=[
            pl.BlockSpec((4, _BLK, 16), lambda i: (0, i, 0)),
            full((1, 64)), full((64, 64)), full((1, 64)),
            full((64, 8)), full((64, 1)), full((8, 64)), full((64, 8)),
            full((8, 7)), full((7, 1)), full((7, 1)),
        ],
        out_specs=[row(16), row(16)],
        out_shape=[
            jax.ShapeDtypeStruct((_N, 16), jnp.float32),
            jax.ShapeDtypeStruct((_N, 16), jnp.float32),
        ],
    )(nd, b1, bm_mat, bmt, a1_mat, v2, r_mat, s_mat, w3, al3, ar3)


def _fin_body(acc_ref, b3_ref, out_ref):
    a = acc_ref[...][0] + acc_ref[...][1]
    out_ref[...] = a[:, :7] / jnp.maximum(a[:, 7:8], 1e-9) + b3_ref[...]


def _final(acc3, b3):
    return pl.pallas_call(
        _fin_body,
        grid=(_N // _BLK,),
        in_specs=[
            pl.BlockSpec((2, _BLK, 16), lambda i: (0, i, 0)),
            pl.BlockSpec((1, 7), lambda i: (0, 0)),
        ],
        out_specs=pl.BlockSpec((_BLK, 7), lambda i: (i, 0)),
        out_shape=jax.ShapeDtypeStruct((_N, 7), jnp.float32),
    )(acc3, b3.reshape(1, 7))


def kernel(x, edge_index, W1, attn_l1, attn_r1, b1, Wm, bm, a, W3,
           attn_l3, attn_r3, b3):
    n = x.shape[0]
    src = edge_index[0]
    dst = edge_index[1]

    npad = _EP - _E
    src_p = jnp.concatenate([src, jnp.zeros((npad,), jnp.int32)])
    dst_p = jnp.concatenate(
        [dst, _N + (jnp.arange(npad, dtype=jnp.int32) % 128)])

    head_of_col = 2 * (jnp.arange(64) // 16) + (jnp.arange(64) % 16) // 8
    sel = (jnp.arange(8)[:, None] == head_of_col[None, :]).astype(jnp.float32)
    eye_rep = jnp.repeat(jnp.eye(8, dtype=jnp.float32), 8, axis=0)
    al_mat = eye_rep * attn_l1.reshape(-1, 1)
    ar_mat = eye_rep * attn_r1.reshape(-1, 1)
    alx = al_mat @ sel
    arx = ar_mat @ sel
    bm_mat = jnp.kron(jnp.eye(8, dtype=jnp.float32), Wm.T)
    bmt = jnp.tile(bm, 8).reshape(1, 64)
    a1_mat = eye_rep * jnp.tile(a[:8, 0], 8).reshape(-1, 1)
    v2 = (jnp.tile(a[8:, 0], 8) / 8.0).reshape(64, 1)
    r_mat = jnp.repeat(jnp.eye(8, dtype=jnp.float32), 8, axis=1)
    s_mat = jnp.tile(jnp.eye(8, dtype=jnp.float32), (8, 1))

    fse, erx = _layer1(x, W1, alx, arx)
    fse = fse.reshape(4 * n, 32)
    erx = erx.reshape(4 * n, 16)

    nd = _edge1(src_p, dst_p, fse, erx)

    t3s, t3d = _mhi_layer(nd, b1.reshape(1, 64), bm_mat, bmt, a1_mat,
                          v2, r_mat, s_mat, W3, attn_l3.reshape(7, 1),
                          attn_r3.reshape(7, 1))

    acc3 = _edge3(src_p, dst_p, t3s, t3d)

    return _final(acc3, b3)


# speedup vs baseline: 1.5248x; 1.5248x over previous
import functools

import jax
import jax.numpy as jnp
from jax import lax
from jax.experimental import pallas as pl
from jax.experimental.pallas import tpu as pltpu
from jax.experimental.pallas import tpu_sc as plsc

_N = 50000
_E = 800000
_CH = 128
_EP = 819200
_NCH = _EP // _CH
_ACC = 50176
_ZSTRIPE = _ACC // 16
_ZB = 784
_WSTRIPE = 3128
_NOUT = 50048
_BLK = 1000


def _l1_body(x_ref, w_ref, alx_ref, arx_ref, elx_ref, erx_ref, fx_ref):
    f = jnp.dot(x_ref[...], w_ref[...], preferred_element_type=jnp.float32)
    elxx = jnp.dot(f, alx_ref[...], preferred_element_type=jnp.float32)
    erxx = jnp.dot(f, arx_ref[...], preferred_element_type=jnp.float32)
    for q in range(4):
        elx_ref[q] = elxx[:, 16 * q:16 * q + 16]
        erx_ref[q] = erxx[:, 16 * q:16 * q + 16]
        fx_ref[q] = f[:, 16 * q:16 * q + 16]


def _layer1(x, w1, alx, arx):
    n = x.shape[0]
    k = x.shape[1]
    out3 = jax.ShapeDtypeStruct((4, n, 16), jnp.float32)
    return pl.pallas_call(
        _l1_body,
        grid=(n // _BLK,),
        in_specs=[
            pl.BlockSpec((_BLK, k), lambda i: (i, 0)),
            pl.BlockSpec((k, 64), lambda i: (0, 0)),
            pl.BlockSpec((64, 64), lambda i: (0, 0)),
            pl.BlockSpec((64, 64), lambda i: (0, 0)),
        ],
        out_specs=[pl.BlockSpec((4, _BLK, 16), lambda i: (0, i, 0))] * 3,
        out_shape=[out3, out3, out3],
    )(x, w1, alx, arx)


def _edge1_body(src_hbm, dst_hbm, elx, erx, fx, nd_out,
                snum, sden, wbn, wbd, nbuf,
                sraw0, sraw1, draw0, draw1,
                sq0, sq1, dq0, dq1,
                dss0, dss1, six0, six1,
                gs0, gs1, gd0, gd1, gf0, gf1,
                upd0, upd1, exb0, exb1,
                sl0, sl1, sg0, sg1, ss0, ss1):
    cid = lax.axis_index("c")
    sid = lax.axis_index("s")
    sraw = [sraw0, sraw1]
    draw = [draw0, draw1]
    sq = [sq0, sq1]
    dq = [dq0, dq1]
    dss = [dss0, dss1]
    six = [six0, six1]
    gs = [gs0, gs1]
    gd = [gd0, gd1]
    gf = [gf0, gf1]
    upd = [upd0, upd1]
    exb = [exb0, exb1]
    sl = [sl0, sl1]
    sg = [sg0, sg1]
    ss = [ss0, ss1]
    nch = _NCH // 16

    def zero_upd0(i, _):
        upd0[i] = jnp.zeros((16,), jnp.float32)
        return 0

    def issue_linear(ch, p):
        off = ch * _CH
        pltpu.make_async_copy(src_hbm.at[pl.ds(off, _CH)], sraw[p], sl[p]).start()
        pltpu.make_async_copy(dst_hbm.at[pl.ds(off, _CH)], draw[p], sl[p]).start()

    def wait_linear(p):
        pltpu.make_async_copy(src_hbm.at[pl.ds(0, _CH)], sraw[p], sl[p]).wait()
        pltpu.make_async_copy(dst_hbm.at[pl.ds(0, _CH)], draw[p], sl[p]).wait()

    def adjust(p, slot, qn):
        for l in range(8):
            s_ = pl.ds(l * 16, 16)
            s = sraw[p][s_]
            d = draw[p][s_]
            sq[p][s_] = s + qn
            dq[p][s_] = jnp.minimum(d, _N - 1) + qn
            dss[slot][s_] = d

    def issue_gathers(p):
        pltpu.make_async_copy(elx.at[sq[p]], gs[p], sg[p]).start()
        pltpu.make_async_copy(erx.at[dq[p]], gd[p], sg[p]).start()
        pltpu.make_async_copy(fx.at[sq[p]], gf[p], sg[p]).start()

    def wait_gathers(p):
        pltpu.make_async_copy(elx.at[sq[p]], gs[p], sg[p]).wait()
        pltpu.make_async_copy(erx.at[dq[p]], gd[p], sg[p]).wait()
        pltpu.make_async_copy(fx.at[sq[p]], gf[p], sg[p]).wait()

    def compute(p):
        def rowblk(i, _):
            for u in range(8):
                r = i * 8 + u
                e = gs[p][r] + gd[p][r]
                ex = jnp.exp(jnp.maximum(e, e * 0.2))
                upd[p][r] = gf[p][r] * ex
                exb[p][r] = ex
            return 0

        lax.fori_loop(0, _CH // 8, rowblk, 0)

    def copy_six(p):
        for l in range(8):
            s_ = pl.ds(l * 16, 16)
            six[p][s_] = dss[p][s_]

    def issue_scatter(p):
        pltpu.make_async_copy(upd[p], snum.at[six[p]], ss[p]).start(add=True)
        pltpu.make_async_copy(exb[p], sden.at[six[p]], ss[p]).start(add=True)

    def wait_scatter(p):
        pltpu.make_async_copy(upd[p], snum.at[six[p]], ss[p]).wait()
        pltpu.make_async_copy(exb[p], sden.at[six[p]], ss[p]).wait()

    base = sid * nch
    zbase = sid * _ZSTRIPE
    wb = sid * _WSTRIPE

    def nrow(i, _):
        nbuf[i] = wbn[i] / jnp.maximum(wbd[i], 1e-9)
        return 0

    def step(i, p, qn):
        @pl.when(i + 2 < nch)
        def _():
            issue_linear(base + i + 2, p)

        @pl.when(i + 1 < nch)
        def _():
            wait_linear(1 - p)
            adjust(1 - p, 1 - p, qn)
            issue_gathers(1 - p)

        wait_gathers(p)

        @pl.when(i >= 2)
        def _():
            wait_scatter(p)

        compute(p)
        copy_six(p)
        issue_scatter(p)

    def round_body(g, _):
        qn = (cid * 2 + g) * _N
        q = cid * 2 + g
        lax.fori_loop(0, _CH, zero_upd0, 0)

        def zrow(z, c):
            pltpu.sync_copy(upd0, snum.at[pl.ds(zbase + z * _CH, _CH)])
            pltpu.sync_copy(upd0, sden.at[pl.ds(zbase + z * _CH, _CH)])
            return c

        lax.fori_loop(0, 24, zrow, 0)
        pltpu.sync_copy(upd0.at[pl.ds(0, 64)],
                        snum.at[pl.ds(zbase + 24 * _CH, 64)])
        pltpu.sync_copy(upd0.at[pl.ds(0, 64)],
                        sden.at[pl.ds(zbase + 24 * _CH, 64)])
        plsc.subcore_barrier()

        issue_linear(base, 0)
        wait_linear(0)
        adjust(0, 0, qn)
        issue_gathers(0)
        issue_linear(base + 1, 1)

        def outer(i2, c):
            step(i2 * 2, 0, qn)
            step(i2 * 2 + 1, 1, qn)
            return c

        lax.fori_loop(0, nch // 2, outer, 0)
        wait_scatter(0)
        wait_scatter(1)
        plsc.subcore_barrier()

        def wrow(z, c):
            roff = wb + z * _CH
            pltpu.sync_copy(snum.at[pl.ds(roff, _CH)], wbn)
            pltpu.sync_copy(sden.at[pl.ds(roff, _CH)], wbd)
            lax.fori_loop(0, _CH, nrow, 0)
            pltpu.sync_copy(nbuf, nd_out.at[q, pl.ds(roff, _CH)])
            return c

        lax.fori_loop(0, 24, wrow, 0)
        roff = wb + 24 * _CH
        pltpu.sync_copy(snum.at[pl.ds(roff, 56)], wbn.at[pl.ds(0, 56)])
        pltpu.sync_copy(sden.at[pl.ds(roff, 56)], wbd.at[pl.ds(0, 56)])
        lax.fori_loop(0, 56, nrow, 0)
        pltpu.sync_copy(nbuf.at[pl.ds(0, 56)], nd_out.at[q, pl.ds(roff, 56)])
        plsc.subcore_barrier()
        return 0

    lax.fori_loop(0, 2, round_body, 0)


def _edge1(src_p, dst_p, elx, erx, fx):
    mesh = plsc.VectorSubcoreMesh(core_axis_name="c", subcore_axis_name="s")
    f32 = jnp.float32
    i32 = jnp.int32
    idxbuf = pltpu.VMEM((_CH,), i32)
    rowbuf = pltpu.VMEM((_CH, 16), f32)
    kern = pl.kernel(
        _edge1_body,
        out_type=jax.ShapeDtypeStruct((4, _NOUT, 16), f32),
        mesh=mesh,
        compiler_params=pltpu.CompilerParams(use_tc_tiling_on_sc=False),
        scratch_types=[
            pltpu.VMEM_SHARED((_ACC, 16), f32),
            pltpu.VMEM_SHARED((_ACC, 16), f32),
            rowbuf, rowbuf, rowbuf,
            idxbuf, idxbuf, idxbuf, idxbuf,
            idxbuf, idxbuf, idxbuf, idxbuf,
            idxbuf, idxbuf, idxbuf, idxbuf,
            rowbuf, rowbuf, rowbuf, rowbuf, rowbuf, rowbuf,
            rowbuf, rowbuf, rowbuf, rowbuf,
            pltpu.SemaphoreType.DMA, pltpu.SemaphoreType.DMA,
            pltpu.SemaphoreType.DMA, pltpu.SemaphoreType.DMA,
            pltpu.SemaphoreType.DMA, pltpu.SemaphoreType.DMA,
        ],
    )
    return kern(src_p, dst_p, elx, erx, fx)


def _edge3_body(src_hbm, dst_hbm, t3s, t3d, out3,
                acc,
                sraw0, sraw1, draw0, draw1,
                sq0, sq1, dq0, dq1,
                dss0, dss1,
                gs0, gs1, gd0, gd1, upd0, upd1,
                sl0, sl1, sg0, sg1):
    cid = lax.axis_index("c")
    sid = lax.axis_index("s")
    sraw = [sraw0, sraw1]
    draw = [draw0, draw1]
    sq = [sq0, sq1]
    dq = [dq0, dq1]
    dss = [dss0, dss1]
    gs = [gs0, gs1]
    gd = [gd0, gd1]
    upd = [upd0, upd1]
    sl = [sl0, sl1]
    sg = [sg0, sg1]
    nch = _NCH // 32
    wid = cid * 16 + sid

    def zero_upd0(i, _):
        upd0[i] = jnp.zeros((16,), jnp.float32)
        return 0

    lax.fori_loop(0, _CH, zero_upd0, 0)

    def issue_linear(ch, p):
        off = ch * _CH
        pltpu.make_async_copy(src_hbm.at[pl.ds(off, _CH)], sraw[p], sl[p]).start()
        pltpu.make_async_copy(dst_hbm.at[pl.ds(off, _CH)], draw[p], sl[p]).start()

    def wait_linear(p):
        pltpu.make_async_copy(src_hbm.at[pl.ds(0, _CH)], sraw[p], sl[p]).wait()
        pltpu.make_async_copy(dst_hbm.at[pl.ds(0, _CH)], draw[p], sl[p]).wait()

    def adjust(p, slot):
        for l in range(8):
            s_ = pl.ds(l * 16, 16)
            d = draw[p][s_]
            sq[p][s_] = sraw[p][s_]
            dq[p][s_] = jnp.minimum(d, _N - 1)
            dss[slot][s_] = d

    def issue_gathers(p):
        pltpu.make_async_copy(t3s.at[sq[p]], gs[p], sg[p]).start()
        pltpu.make_async_copy(t3d.at[dq[p]], gd[p], sg[p]).start()

    def wait_gathers(p):
        pltpu.make_async_copy(t3s.at[sq[p]], gs[p], sg[p]).wait()
        pltpu.make_async_copy(t3d.at[dq[p]], gd[p], sg[p]).wait()

    def compute(p):
        def rowblk(i, _):
            for u in range(4):
                r = i * 4 + u
                s = gs[p][r]
                e = s + gd[p][r]
                ex = jnp.exp(jnp.maximum(e, e * 0.2))
                exv = jnp.broadcast_to(ex[8], (16,))
                upd[p][r] = s * exv
            return 0

        lax.fori_loop(0, _CH // 4, rowblk, 0)

    zbase = sid * _ZSTRIPE

    def zrow(z, c):
        pltpu.sync_copy(upd0, acc.at[pl.ds(zbase + z * _CH, _CH)])
        return c

    lax.fori_loop(0, 24, zrow, 0)
    pltpu.sync_copy(upd0.at[pl.ds(0, 64)],
                    acc.at[pl.ds(zbase + 24 * _CH, 64)])
    plsc.subcore_barrier()

    base = wid * nch

    def step(i, p):
        @pl.when(i + 2 < nch)
        def _():
            issue_linear(base + i + 2, p)

        @pl.when(i + 1 < nch)
        def _():
            wait_linear(1 - p)
            adjust(1 - p, 1 - p)
            issue_gathers(1 - p)

        wait_gathers(p)
        compute(p)
        pltpu.sync_copy(upd[p], acc.at[dss[p]], add=True)

    issue_linear(base, 0)
    wait_linear(0)
    adjust(0, 0)
    issue_gathers(0)
    issue_linear(base + 1, 1)

    def outer(i2, c):
        step(i2 * 2, 0)
        step(i2 * 2 + 1, 1)
        return c

    lax.fori_loop(0, nch // 2, outer, 0)
    plsc.subcore_barrier()

    wb = sid * _ZSTRIPE
    pltpu.sync_copy(acc.at[pl.ds(wb, _ZSTRIPE)],
                    out3.at[cid, pl.ds(wb, _ZSTRIPE)])


def _edge3(src_p, dst_p, t3s, t3d):
    mesh = plsc.VectorSubcoreMesh(core_axis_name="c", subcore_axis_name="s")
    f32 = jnp.float32
    i32 = jnp.int32
    idxbuf = pltpu.VMEM((_CH,), i32)
    rowbuf = pltpu.VMEM((_CH, 16), f32)
    kern = pl.kernel(
        _edge3_body,
        out_type=jax.ShapeDtypeStruct((2, _ACC, 16), f32),
        mesh=mesh,
        compiler_params=pltpu.CompilerParams(use_tc_tiling_on_sc=False),
        scratch_types=[
            pltpu.VMEM_SHARED((_ACC, 16), f32),
            idxbuf, idxbuf, idxbuf, idxbuf,
            idxbuf, idxbuf, idxbuf, idxbuf,
            idxbuf, idxbuf,
            rowbuf, rowbuf, rowbuf, rowbuf, rowbuf, rowbuf,
            pltpu.SemaphoreType.DMA, pltpu.SemaphoreType.DMA,
            pltpu.SemaphoreType.DMA, pltpu.SemaphoreType.DMA,
        ],
    )
    return kern(src_p, dst_p, t3s, t3d)


def _mhi_body(nd_ref, b1_ref, bm_mat_ref, bmt_ref, a1_ref, v2_ref,
              r_ref, s_ref, w3_ref, al3_ref, ar3_ref, t3s_ref, t3d_ref):
    nd = nd_ref[...]
    rst = jnp.concatenate([nd[0], nd[1], nd[2], nd[3]], axis=-1)
    h = jnp.maximum(rst + b1_ref[...], 0.0)
    x2 = jnp.dot(h, bm_mat_ref[...], preferred_element_type=jnp.float32)
    x2 = x2 + bmt_ref[...]
    s1 = jnp.dot(x2, a1_ref[...], preferred_element_type=jnp.float32)
    s2 = jnp.dot(x2, v2_ref[...], preferred_element_type=jnp.float32)
    e = jnp.maximum(s1 + s2, 0.0)
    m = jnp.max(e, axis=1, keepdims=True)
    ex = jnp.exp(e - m)
    alpha = ex / jnp.sum(ex, axis=1, keepdims=True)
    alf = jnp.dot(alpha, r_ref[...], preferred_element_type=jnp.float32)
    h2 = jnp.dot(h * alf, s_ref[...], preferred_element_type=jnp.float32)
    f3 = jnp.dot(h2, w3_ref[...], preferred_element_type=jnp.float32)
    el3 = jnp.dot(f3, al3_ref[...], preferred_element_type=jnp.float32)
    er3 = jnp.dot(f3, ar3_ref[...], preferred_element_type=jnp.float32)
    blk = f3.shape[0]
    one = jnp.ones((blk, 1), jnp.float32)
    t3s_ref[...] = jnp.concatenate(
        [f3, one, jnp.broadcast_to(el3, (blk, 8))], axis=1)
    t3d_ref[...] = jnp.concatenate(
        [jnp.zeros((blk, 8), jnp.float32), jnp.broadcast_to(er3, (blk, 8))],
        axis=1)


def _mhi_layer(nd, b1, bm_mat, bmt, a1_mat, v2, r_mat, s_mat, w3, al3, ar3):
    full = lambda shape: pl.BlockSpec(shape, lambda i: tuple(0 for _ in shape))
    row = lambda c: pl.BlockSpec((_BLK, c), lambda i: (i, 0))
    return pl.pallas_call(
        _mhi_body,
        grid=(_N // _BLK,),
        in_specs=[
            pl.BlockSpec((4, _BLK, 16), lambda i: (0, i, 0)),
            full((1, 64)), full((64, 64)), full((1, 64)),
            full((64, 8)), full((64, 1)), full((8, 64)), full((64, 8)),
            full((8, 7)), full((7, 1)), full((7, 1)),
        ],
        out_specs=[row(16), row(16)],
        out_shape=[
            jax.ShapeDtypeStruct((_N, 16), jnp.float32),
            jax.ShapeDtypeStruct((_N, 16), jnp.float32),
        ],
    )(nd, b1, bm_mat, bmt, a1_mat, v2, r_mat, s_mat, w3, al3, ar3)


def _fin_body(acc_ref, b3_ref, out_ref):
    a = acc_ref[...][0] + acc_ref[...][1]
    out_ref[...] = a[:, :7] / jnp.maximum(a[:, 7:8], 1e-9) + b3_ref[...]


def _final(acc3, b3):
    return pl.pallas_call(
        _fin_body,
        grid=(_N // _BLK,),
        in_specs=[
            pl.BlockSpec((2, _BLK, 16), lambda i: (0, i, 0)),
            pl.BlockSpec((1, 7), lambda i: (0, 0)),
        ],
        out_specs=pl.BlockSpec((_BLK, 7), lambda i: (i, 0)),
        out_shape=jax.ShapeDtypeStruct((_N, 7), jnp.float32),
    )(acc3, b3.reshape(1, 7))


def kernel(x, edge_index, W1, attn_l1, attn_r1, b1, Wm, bm, a, W3,
           attn_l3, attn_r3, b3):
    n = x.shape[0]
    src = edge_index[0]
    dst = edge_index[1]

    npad = _EP - _E
    src_p = jnp.concatenate([src, jnp.zeros((npad,), jnp.int32)])
    dst_p = jnp.concatenate(
        [dst, _N + (jnp.arange(npad, dtype=jnp.int32) % 128)])

    head_of_col = 2 * (jnp.arange(64) // 16) + (jnp.arange(64) % 16) // 8
    sel = (jnp.arange(8)[:, None] == head_of_col[None, :]).astype(jnp.float32)
    eye_rep = jnp.repeat(jnp.eye(8, dtype=jnp.float32), 8, axis=0)
    al_mat = eye_rep * attn_l1.reshape(-1, 1)
    ar_mat = eye_rep * attn_r1.reshape(-1, 1)
    alx = al_mat @ sel
    arx = ar_mat @ sel
    bm_mat = jnp.kron(jnp.eye(8, dtype=jnp.float32), Wm.T)
    bmt = jnp.tile(bm, 8).reshape(1, 64)
    a1_mat = eye_rep * jnp.tile(a[:8, 0], 8).reshape(-1, 1)
    v2 = (jnp.tile(a[8:, 0], 8) / 8.0).reshape(64, 1)
    r_mat = jnp.repeat(jnp.eye(8, dtype=jnp.float32), 8, axis=1)
    s_mat = jnp.tile(jnp.eye(8, dtype=jnp.float32), (8, 1))

    elx, erx, fx = _layer1(x, W1, alx, arx)
    elx = elx.reshape(4 * n, 16)
    erx = erx.reshape(4 * n, 16)
    fx = fx.reshape(4 * n, 16)

    nd = _edge1(src_p, dst_p, elx, erx, fx)

    t3s, t3d = _mhi_layer(nd, b1.reshape(1, 64), bm_mat, bmt, a1_mat,
                          v2, r_mat, s_mat, W3, attn_l3.reshape(7, 1),
                          attn_r3.reshape(7, 1))

    acc3 = _edge3(src_p, dst_p, t3s, t3d)

    return _final(acc3, b3)


# speedup vs baseline: 1.5291x; 1.0028x over previous
import functools

import jax
import jax.numpy as jnp
from jax import lax
from jax.experimental import pallas as pl
from jax.experimental.pallas import tpu as pltpu
from jax.experimental.pallas import tpu_sc as plsc

_N = 50000
_E = 800000
_CH = 128
_EP = 819200
_NCH = _EP // _CH
_ACC = 50176
_ZSTRIPE = _ACC // 16
_ZB = 784
_WSTRIPE = 3128
_NOUT = 50048
_BLK = 1000


def _l1_body(x_ref, w_ref, alx_ref, arx_ref, elx_ref, erx_ref, fx_ref):
    f = jnp.dot(x_ref[...], w_ref[...], preferred_element_type=jnp.float32)
    elxx = jnp.dot(f, alx_ref[...], preferred_element_type=jnp.float32)
    erxx = jnp.dot(f, arx_ref[...], preferred_element_type=jnp.float32)
    for q in range(4):
        elx_ref[q] = elxx[:, 16 * q:16 * q + 16]
        erx_ref[q] = erxx[:, 16 * q:16 * q + 16]
        fx_ref[q] = f[:, 16 * q:16 * q + 16]


def _layer1(x, w1, alx, arx):
    n = x.shape[0]
    k = x.shape[1]
    out3 = jax.ShapeDtypeStruct((4, n, 16), jnp.float32)
    return pl.pallas_call(
        _l1_body,
        grid=(n // _BLK,),
        in_specs=[
            pl.BlockSpec((_BLK, k), lambda i: (i, 0)),
            pl.BlockSpec((k, 64), lambda i: (0, 0)),
            pl.BlockSpec((64, 64), lambda i: (0, 0)),
            pl.BlockSpec((64, 64), lambda i: (0, 0)),
        ],
        out_specs=[pl.BlockSpec((4, _BLK, 16), lambda i: (0, i, 0))] * 3,
        out_shape=[out3, out3, out3],
    )(x, w1, alx, arx)


def _edge1_body(src_hbm, dst_hbm, elx, erx, fx, nd_out,
                snum, sden, wbn, wbd, nbuf,
                sraw0, sraw1, draw0, draw1,
                sq0, sq1, dq0, dq1,
                dss0, dss1, six0, six1,
                gs0, gs1, gd0, gd1, gf0, gf1,
                upd0, upd1, exb0, exb1,
                sl0, sl1, sg0, sg1, ss0, ss1):
    cid = lax.axis_index("c")
    sid = lax.axis_index("s")
    sraw = [sraw0, sraw1]
    draw = [draw0, draw1]
    sq = [sq0, sq1]
    dq = [dq0, dq1]
    dss = [dss0, dss1]
    six = [six0, six1]
    gs = [gs0, gs1]
    gd = [gd0, gd1]
    gf = [gf0, gf1]
    upd = [upd0, upd1]
    exb = [exb0, exb1]
    sl = [sl0, sl1]
    sg = [sg0, sg1]
    ss = [ss0, ss1]
    nch = _NCH // 16

    def zero_upd0(i, _):
        upd0[i] = jnp.zeros((16,), jnp.float32)
        return 0

    def issue_linear(ch, p):
        off = ch * _CH
        pltpu.make_async_copy(src_hbm.at[pl.ds(off, _CH)], sraw[p], sl[p]).start()
        pltpu.make_async_copy(dst_hbm.at[pl.ds(off, _CH)], draw[p], sl[p]).start()

    def wait_linear(p):
        pltpu.make_async_copy(src_hbm.at[pl.ds(0, _CH)], sraw[p], sl[p]).wait()
        pltpu.make_async_copy(dst_hbm.at[pl.ds(0, _CH)], draw[p], sl[p]).wait()

    def adjust(p, slot, qn):
        for l in range(8):
            s_ = pl.ds(l * 16, 16)
            s = sraw[p][s_]
            d = draw[p][s_]
            sq[p][s_] = s + qn
            dq[p][s_] = jnp.minimum(d, _N - 1) + qn
            dss[slot][s_] = d

    def issue_gathers(p):
        pltpu.make_async_copy(elx.at[sq[p]], gs[p], sg[p]).start()
        pltpu.make_async_copy(erx.at[dq[p]], gd[p], sg[p]).start()
        pltpu.make_async_copy(fx.at[sq[p]], gf[p], sg[p]).start()

    def wait_gathers(p):
        pltpu.make_async_copy(elx.at[sq[p]], gs[p], sg[p]).wait()
        pltpu.make_async_copy(erx.at[dq[p]], gd[p], sg[p]).wait()
        pltpu.make_async_copy(fx.at[sq[p]], gf[p], sg[p]).wait()

    def compute(p):
        def rowblk(i, _):
            for u in range(8):
                r = i * 8 + u
                e = gs[p][r] + gd[p][r]
                ex = jnp.exp(jnp.maximum(e, e * 0.2))
                upd[p][r] = gf[p][r] * ex
                exb[p][r] = ex
            return 0

        lax.fori_loop(0, _CH // 8, rowblk, 0)

    def copy_six(p):
        for l in range(8):
            s_ = pl.ds(l * 16, 16)
            six[p][s_] = dss[p][s_]

    def issue_scatter(p):
        pltpu.make_async_copy(upd[p], snum.at[six[p]], ss[p]).start(add=True)
        pltpu.make_async_copy(exb[p], sden.at[six[p]], ss[p]).start(add=True)

    def wait_scatter(p):
        pltpu.make_async_copy(upd[p], snum.at[six[p]], ss[p]).wait()
        pltpu.make_async_copy(exb[p], sden.at[six[p]], ss[p]).wait()

    base = sid * nch
    zbase = sid * _ZSTRIPE
    wb = sid * _WSTRIPE

    def nrow(i, _):
        nbuf[i] = wbn[i] / jnp.maximum(wbd[i], 1e-9)
        return 0

    def step(i, p, qn):
        @pl.when(i + 2 < nch)
        def _():
            issue_linear(base + i + 2, p)

        @pl.when(i + 1 < nch)
        def _():
            wait_linear(1 - p)
            adjust(1 - p, 1 - p, qn)
            issue_gathers(1 - p)

        wait_gathers(p)

        @pl.when(i >= 2)
        def _():
            wait_scatter(p)

        compute(p)
        copy_six(p)
        issue_scatter(p)

    def round_body(g, _):
        qn = (cid * 2 + g) * _N
        q = cid * 2 + g
        lax.fori_loop(0, _CH, zero_upd0, 0)

        def zrow(z, c):
            pltpu.sync_copy(upd0, snum.at[pl.ds(zbase + z * _CH, _CH)])
            pltpu.sync_copy(upd0, sden.at[pl.ds(zbase + z * _CH, _CH)])
            return c

        lax.fori_loop(0, 24, zrow, 0)
        pltpu.sync_copy(upd0.at[pl.ds(0, 64)],
                        snum.at[pl.ds(zbase + 24 * _CH, 64)])
        pltpu.sync_copy(upd0.at[pl.ds(0, 64)],
                        sden.at[pl.ds(zbase + 24 * _CH, 64)])
        plsc.subcore_barrier()

        issue_linear(base, 0)
        wait_linear(0)
        adjust(0, 0, qn)
        issue_gathers(0)
        issue_linear(base + 1, 1)

        def outer(i2, c):
            step(i2 * 2, 0, qn)
            step(i2 * 2 + 1, 1, qn)
            return c

        lax.fori_loop(0, nch // 2, outer, 0)
        wait_scatter(0)
        wait_scatter(1)
        plsc.subcore_barrier()

        def wrow(z, c):
            roff = wb + z * _CH
            pltpu.sync_copy(snum.at[pl.ds(roff, _CH)], wbn)
            pltpu.sync_copy(sden.at[pl.ds(roff, _CH)], wbd)
            lax.fori_loop(0, _CH, nrow, 0)
            pltpu.sync_copy(nbuf, nd_out.at[q, pl.ds(roff, _CH)])
            return c

        lax.fori_loop(0, 24, wrow, 0)
        roff = wb + 24 * _CH
        pltpu.sync_copy(snum.at[pl.ds(roff, 56)], wbn.at[pl.ds(0, 56)])
        pltpu.sync_copy(sden.at[pl.ds(roff, 56)], wbd.at[pl.ds(0, 56)])
        lax.fori_loop(0, 56, nrow, 0)
        pltpu.sync_copy(nbuf.at[pl.ds(0, 56)], nd_out.at[q, pl.ds(roff, 56)])
        plsc.subcore_barrier()
        return 0

    lax.fori_loop(0, 2, round_body, 0)


def _edge1(src_p, dst_p, elx, erx, fx):
    mesh = plsc.VectorSubcoreMesh(core_axis_name="c", subcore_axis_name="s")
    f32 = jnp.float32
    i32 = jnp.int32
    idxbuf = pltpu.VMEM((_CH,), i32)
    rowbuf = pltpu.VMEM((_CH, 16), f32)
    kern = pl.kernel(
        _edge1_body,
        out_type=jax.ShapeDtypeStruct((4, _NOUT, 16), f32),
        mesh=mesh,
        compiler_params=pltpu.CompilerParams(use_tc_tiling_on_sc=False),
        scratch_types=[
            pltpu.VMEM_SHARED((_ACC, 16), f32),
            pltpu.VMEM_SHARED((_ACC, 16), f32),
            rowbuf, rowbuf, rowbuf,
            idxbuf, idxbuf, idxbuf, idxbuf,
            idxbuf, idxbuf, idxbuf, idxbuf,
            idxbuf, idxbuf, idxbuf, idxbuf,
            rowbuf, rowbuf, rowbuf, rowbuf, rowbuf, rowbuf,
            rowbuf, rowbuf, rowbuf, rowbuf,
            pltpu.SemaphoreType.DMA, pltpu.SemaphoreType.DMA,
            pltpu.SemaphoreType.DMA, pltpu.SemaphoreType.DMA,
            pltpu.SemaphoreType.DMA, pltpu.SemaphoreType.DMA,
        ],
    )
    return kern(src_p, dst_p, elx, erx, fx)


def _edge3_body(src_hbm, dst_hbm, t3s, t3d, out3,
                acc,
                sraw0, sraw1, draw0, draw1,
                sq0, sq1, dq0, dq1,
                dss0, dss1, six0, six1,
                gs0, gs1, gd0, gd1, upd0, upd1,
                sl0, sl1, sg0, sg1, ss0, ss1):
    cid = lax.axis_index("c")
    sid = lax.axis_index("s")
    sraw = [sraw0, sraw1]
    draw = [draw0, draw1]
    sq = [sq0, sq1]
    dq = [dq0, dq1]
    dss = [dss0, dss1]
    six = [six0, six1]
    gs = [gs0, gs1]
    gd = [gd0, gd1]
    upd = [upd0, upd1]
    sl = [sl0, sl1]
    sg = [sg0, sg1]
    ss = [ss0, ss1]
    nch = _NCH // 32
    wid = cid * 16 + sid

    def zero_upd0(i, _):
        upd0[i] = jnp.zeros((16,), jnp.float32)
        return 0

    lax.fori_loop(0, _CH, zero_upd0, 0)

    def issue_linear(ch, p):
        off = ch * _CH
        pltpu.make_async_copy(src_hbm.at[pl.ds(off, _CH)], sraw[p], sl[p]).start()
        pltpu.make_async_copy(dst_hbm.at[pl.ds(off, _CH)], draw[p], sl[p]).start()

    def wait_linear(p):
        pltpu.make_async_copy(src_hbm.at[pl.ds(0, _CH)], sraw[p], sl[p]).wait()
        pltpu.make_async_copy(dst_hbm.at[pl.ds(0, _CH)], draw[p], sl[p]).wait()

    def adjust(p, slot):
        for l in range(8):
            s_ = pl.ds(l * 16, 16)
            d = draw[p][s_]
            sq[p][s_] = sraw[p][s_]
            dq[p][s_] = jnp.minimum(d, _N - 1)
            dss[slot][s_] = d

    def issue_gathers(p):
        pltpu.make_async_copy(t3s.at[sq[p]], gs[p], sg[p]).start()
        pltpu.make_async_copy(t3d.at[dq[p]], gd[p], sg[p]).start()

    def wait_gathers(p):
        pltpu.make_async_copy(t3s.at[sq[p]], gs[p], sg[p]).wait()
        pltpu.make_async_copy(t3d.at[dq[p]], gd[p], sg[p]).wait()

    def compute(p):
        def rowblk(i, _):
            for u in range(4):
                r = i * 4 + u
                s = gs[p][r]
                e = s + gd[p][r]
                ex = jnp.exp(jnp.maximum(e, e * 0.2))
                exv = jnp.broadcast_to(ex[8], (16,))
                upd[p][r] = s * exv
            return 0

        lax.fori_loop(0, _CH // 4, rowblk, 0)

    def copy_six(p):
        for l in range(8):
            s_ = pl.ds(l * 16, 16)
            six[p][s_] = dss[p][s_]

    def issue_scatter(p):
        pltpu.make_async_copy(upd[p], acc.at[six[p]], ss[p]).start(add=True)

    def wait_scatter(p):
        pltpu.make_async_copy(upd[p], acc.at[six[p]], ss[p]).wait()

    zbase = sid * _ZSTRIPE

    def zrow(z, c):
        pltpu.sync_copy(upd0, acc.at[pl.ds(zbase + z * _CH, _CH)])
        return c

    lax.fori_loop(0, 24, zrow, 0)
    pltpu.sync_copy(upd0.at[pl.ds(0, 64)],
                    acc.at[pl.ds(zbase + 24 * _CH, 64)])
    plsc.subcore_barrier()

    base = wid * nch

    def step(i, p):
        @pl.when(i + 2 < nch)
        def _():
            issue_linear(base + i + 2, p)

        @pl.when(i + 1 < nch)
        def _():
            wait_linear(1 - p)
            adjust(1 - p, 1 - p)
            issue_gathers(1 - p)

        wait_gathers(p)

        @pl.when(i >= 2)
        def _():
            wait_scatter(p)

        compute(p)
        copy_six(p)
        issue_scatter(p)

    issue_linear(base, 0)
    wait_linear(0)
    adjust(0, 0)
    issue_gathers(0)
    issue_linear(base + 1, 1)

    def outer(i2, c):
        step(i2 * 2, 0)
        step(i2 * 2 + 1, 1)
        return c

    lax.fori_loop(0, nch // 2, outer, 0)
    wait_scatter(0)
    wait_scatter(1)
    plsc.subcore_barrier()

    wb = sid * _ZSTRIPE
    pltpu.sync_copy(acc.at[pl.ds(wb, _ZSTRIPE)],
                    out3.at[cid, pl.ds(wb, _ZSTRIPE)])


def _edge3(src_p, dst_p, t3s, t3d):
    mesh = plsc.VectorSubcoreMesh(core_axis_name="c", subcore_axis_name="s")
    f32 = jnp.float32
    i32 = jnp.int32
    idxbuf = pltpu.VMEM((_CH,), i32)
    rowbuf = pltpu.VMEM((_CH, 16), f32)
    kern = pl.kernel(
        _edge3_body,
        out_type=jax.ShapeDtypeStruct((2, _ACC, 16), f32),
        mesh=mesh,
        compiler_params=pltpu.CompilerParams(use_tc_tiling_on_sc=False),
        scratch_types=[
            pltpu.VMEM_SHARED((_ACC, 16), f32),
            idxbuf, idxbuf, idxbuf, idxbuf,
            idxbuf, idxbuf, idxbuf, idxbuf,
            idxbuf, idxbuf, idxbuf, idxbuf,
            rowbuf, rowbuf, rowbuf, rowbuf, rowbuf, rowbuf,
            pltpu.SemaphoreType.DMA, pltpu.SemaphoreType.DMA,
            pltpu.SemaphoreType.DMA, pltpu.SemaphoreType.DMA,
            pltpu.SemaphoreType.DMA, pltpu.SemaphoreType.DMA,
        ],
    )
    return kern(src_p, dst_p, t3s, t3d)


def _mhi_body(nd_ref, b1_ref, bm_mat_ref, bmt_ref, a1_ref, v2_ref,
              r_ref, s_ref, w3_ref, al3_ref, ar3_ref, t3s_ref, t3d_ref):
    nd = nd_ref[...]
    rst = jnp.concatenate([nd[0], nd[1], nd[2], nd[3]], axis=-1)
    h = jnp.maximum(rst + b1_ref[...], 0.0)
    x2 = jnp.dot(h, bm_mat_ref[...], preferred_element_type=jnp.float32)
    x2 = x2 + bmt_ref[...]
    s1 = jnp.dot(x2, a1_ref[...], preferred_element_type=jnp.float32)
    s2 = jnp.dot(x2, v2_ref[...], preferred_element_type=jnp.float32)
    e = jnp.maximum(s1 + s2, 0.0)
    m = jnp.max(e, axis=1, keepdims=True)
    ex = jnp.exp(e - m)
    alpha = ex / jnp.sum(ex, axis=1, keepdims=True)
    alf = jnp.dot(alpha, r_ref[...], preferred_element_type=jnp.float32)
    h2 = jnp.dot(h * alf, s_ref[...], preferred_element_type=jnp.float32)
    f3 = jnp.dot(h2, w3_ref[...], preferred_element_type=jnp.float32)
    el3 = jnp.dot(f3, al3_ref[...], preferred_element_type=jnp.float32)
    er3 = jnp.dot(f3, ar3_ref[...], preferred_element_type=jnp.float32)
    blk = f3.shape[0]
    one = jnp.ones((blk, 1), jnp.float32)
    t3s_ref[...] = jnp.concatenate(
        [f3, one, jnp.broadcast_to(el3, (blk, 8))], axis=1)
    t3d_ref[...] = jnp.concatenate(
        [jnp.zeros((blk, 8), jnp.float32), jnp.broadcast_to(er3, (blk, 8))],
        axis=1)


def _mhi_layer(nd, b1, bm_mat, bmt, a1_mat, v2, r_mat, s_mat, w3, al3, ar3):
    full = lambda shape: pl.BlockSpec(shape, lambda i: tuple(0 for _ in shape))
    row = lambda c: pl.BlockSpec((_BLK, c), lambda i: (i, 0))
    return pl.pallas_call(
        _mhi_body,
        grid=(_N // _BLK,),
        in_specs=[
            pl.BlockSpec((4, _BLK, 16), lambda i: (0, i, 0)),
            full((1, 64)), full((64, 64)), full((1, 64)),
            full((64, 8)), full((64, 1)), full((8, 64)), full((64, 8)),
            full((8, 7)), full((7, 1)), full((7, 1)),
        ],
        out_specs=[row(16), row(16)],
        out_shape=[
            jax.ShapeDtypeStruct((_N, 16), jnp.float32),
            jax.ShapeDtypeStruct((_N, 16), jnp.float32),
        ],
    )(nd, b1, bm_mat, bmt, a1_mat, v2, r_mat, s_mat, w3, al3, ar3)


def _fin_body(acc_ref, b3_ref, out_ref):
    a = acc_ref[...][0] + acc_ref[...][1]
    out_ref[...] = a[:, :7] / jnp.maximum(a[:, 7:8], 1e-9) + b3_ref[...]


def _final(acc3, b3):
    return pl.pallas_call(
        _fin_body,
        grid=(_N // _BLK,),
        in_specs=[
            pl.BlockSpec((2, _BLK, 16), lambda i: (0, i, 0)),
            pl.BlockSpec((1, 7), lambda i: (0, 0)),
        ],
        out_specs=pl.BlockSpec((_BLK, 7), lambda i: (i, 0)),
        out_shape=jax.ShapeDtypeStruct((_N, 7), jnp.float32),
    )(acc3, b3.reshape(1, 7))


def kernel(x, edge_index, W1, attn_l1, attn_r1, b1, Wm, bm, a, W3,
           attn_l3, attn_r3, b3):
    n = x.shape[0]
    src = edge_index[0]
    dst = edge_index[1]

    npad = _EP - _E
    src_p = jnp.concatenate([src, jnp.zeros((npad,), jnp.int32)])
    dst_p = jnp.concatenate(
        [dst, _N + (jnp.arange(npad, dtype=jnp.int32) % 128)])

    head_of_col = 2 * (jnp.arange(64) // 16) + (jnp.arange(64) % 16) // 8
    sel = (jnp.arange(8)[:, None] == head_of_col[None, :]).astype(jnp.float32)
    eye_rep = jnp.repeat(jnp.eye(8, dtype=jnp.float32), 8, axis=0)
    al_mat = eye_rep * attn_l1.reshape(-1, 1)
    ar_mat = eye_rep * attn_r1.reshape(-1, 1)
    alx = al_mat @ sel
    arx = ar_mat @ sel
    bm_mat = jnp.kron(jnp.eye(8, dtype=jnp.float32), Wm.T)
    bmt = jnp.tile(bm, 8).reshape(1, 64)
    a1_mat = eye_rep * jnp.tile(a[:8, 0], 8).reshape(-1, 1)
    v2 = (jnp.tile(a[8:, 0], 8) / 8.0).reshape(64, 1)
    r_mat = jnp.repeat(jnp.eye(8, dtype=jnp.float32), 8, axis=1)
    s_mat = jnp.tile(jnp.eye(8, dtype=jnp.float32), (8, 1))

    elx, erx, fx = _layer1(x, W1, alx, arx)
    elx = elx.reshape(4 * n, 16)
    erx = erx.reshape(4 * n, 16)
    fx = fx.reshape(4 * n, 16)

    nd = _edge1(src_p, dst_p, elx, erx, fx)

    t3s, t3d = _mhi_layer(nd, b1.reshape(1, 64), bm_mat, bmt, a1_mat,
                          v2, r_mat, s_mat, W3, attn_l3.reshape(7, 1),
                          attn_r3.reshape(7, 1))

    acc3 = _edge3(src_p, dst_p, t3s, t3d)

    return _final(acc3, b3)
